# baseline (speedup 1.0000x reference)
import jax
import jax.numpy as jnp
from jax import lax
from jax.experimental import pallas as pl
from jax.experimental.pallas import tpu as pltpu
from jax.experimental.pallas import tpu_sc as plsc

N_MEM = 100000
N_FEAT = 128
N_BATCH = 1024
MOM = 0.2
IOU_MOM = 0.2
TEMP = 0.05


N_BLK = 5120
GRID = pl.cdiv(N_MEM, N_BLK)
M_ROWS, M_COLS = 800, 125
M_BLK = 40


def _tc_body(x_ref, f_ref, m_ref, out_ref, nf_ref, nm_ref):
    f = f_ref[...]
    acc = lax.dot_general(
        f, x_ref[...], (((1,), (1,)), ((), ())),
        preferred_element_type=jnp.float32)
    out_ref[...] = acc / jnp.float32(TEMP)
    nf_ref[...] = f
    nm_ref[...] = m_ref[...]


_tc_call = pl.pallas_call(
    _tc_body,
    grid=(GRID,),
    in_specs=[
        pl.BlockSpec((N_BATCH, N_FEAT), lambda i: (0, 0)),
        pl.BlockSpec((N_BLK, N_FEAT), lambda i: (i, 0)),
        pl.BlockSpec((M_BLK, M_COLS), lambda i: (jnp.minimum(i, 19), 0)),
    ],
    out_specs=[
        pl.BlockSpec((N_BLK, N_BATCH), lambda i: (i, 0)),
        pl.BlockSpec((N_BLK, N_FEAT), lambda i: (i, 0)),
        pl.BlockSpec((M_BLK, M_COLS), lambda i: (jnp.minimum(i, 19), 0)),
    ],
    out_shape=[
        jax.ShapeDtypeStruct((N_MEM, N_BATCH), jnp.float32),
        jax.ShapeDtypeStruct((N_MEM, N_FEAT), jnp.float32),
        jax.ShapeDtypeStruct((M_ROWS, M_COLS), jnp.float32),
    ],
)


_L = 16
_NW = 32
_BPW = N_BATCH // _NW
_CHUNKS = N_FEAT // _L


def _rsqrt(s):
    bits = plsc.bitcast(s, jnp.int32)
    y = plsc.bitcast(jnp.int32(0x5F3759DF) - lax.shift_right_logical(bits, 1),
                     jnp.float32)
    for _ in range(3):
        y = y * (jnp.float32(1.5) - jnp.float32(0.5) * s * y * y)
    return y


def _sc_body(x_hbm, idx_hbm, iou_hbm, feat_hbm, miou_hbm, updf_hbm, updm_hbm,
             idx_all, table, myidx, jstar, g, u, mvals, ivals, sem):
    cid = lax.axis_index("c")
    sid = lax.axis_index("s")
    wid = sid * 2 + cid
    base = wid * _BPW

    pltpu.sync_copy(idx_hbm, idx_all)
    for v in range(N_BATCH // _L):
        vec = idx_all[pl.ds(v * _L, _L)]
        pos = lax.iota(jnp.int32, _L) + jnp.int32(v * _L)
        plsc.store_scatter(table, [vec], pos)

    pltpu.sync_copy(idx_hbm.at[pl.ds(base, _BPW)], myidx)
    for h in range(_BPW // _L):
        sl = pl.ds(h * _L, _L)
        jstar[sl] = plsc.load_gather(table, [myidx[sl]])

    c0 = pltpu.async_copy(feat_hbm.at[myidx], g, sem)
    c1 = pltpu.async_copy(x_hbm.at[jstar], u, sem)
    c2 = pltpu.async_copy(miou_hbm.at[myidx], mvals, sem)
    c3 = pltpu.async_copy(iou_hbm.at[jstar], ivals, sem)
    c0.wait()
    c1.wait()
    c2.wait()
    c3.wait()

    for r in range(_BPW):
        uvs = []
        acc = jnp.zeros((_L,), jnp.float32)
        for c in range(_CHUNKS):
            sl = pl.ds(c * _L, _L)
            uv = jnp.float32(MOM) * g[r, sl] + jnp.float32(1.0 - MOM) * u[r, sl]
            uvs.append(uv)
            acc = acc + uv * uv
        s = lax.broadcast_in_dim(lax.reduce_sum(acc, axes=(0,)), (_L,), ())
        rs = _rsqrt(s)
        for c in range(_CHUNKS):
            u[r, pl.ds(c * _L, _L)] = uvs[c] * rs

    for h in range(_BPW // _L):
        sl = pl.ds(h * _L, _L)
        mvals[sl] = (jnp.float32(IOU_MOM) * mvals[sl]
                     + jnp.float32(1.0 - IOU_MOM) * ivals[sl])

    s0 = pltpu.async_copy(u, updf_hbm.at[pl.ds(base, _BPW)], sem)
    s1 = pltpu.async_copy(mvals, updm_hbm.at[pl.ds(base, _BPW)], sem)
    s0.wait()
    s1.wait()


_sc_update = pl.kernel(
    _sc_body,
    out_type=(
        jax.ShapeDtypeStruct((N_BATCH, N_FEAT), jnp.float32),
        jax.ShapeDtypeStruct((N_BATCH,), jnp.float32),
    ),
    mesh=plsc.VectorSubcoreMesh(core_axis_name="c", subcore_axis_name="s",
                                num_cores=2, num_subcores=16),
    compiler_params=pltpu.CompilerParams(needs_layout_passes=False),
    scratch_types=[
        pltpu.VMEM((N_BATCH,), jnp.int32),
        pltpu.VMEM((100096,), jnp.int32),
        pltpu.VMEM((_BPW,), jnp.int32),
        pltpu.VMEM((_BPW,), jnp.int32),
        pltpu.VMEM((_BPW, N_FEAT), jnp.float32),
        pltpu.VMEM((_BPW, N_FEAT), jnp.float32),
        pltpu.VMEM((_BPW,), jnp.float32),
        pltpu.VMEM((_BPW,), jnp.float32),
        pltpu.SemaphoreType.DMA,
    ],
)


def _sc_scatter_body(idx_hbm, updf_hbm, updm_hbm, newf_ref, newm_ref,
                     myidx, rows, vals, sem):
    cid = lax.axis_index("c")
    sid = lax.axis_index("s")
    base = (sid * 2 + cid) * _BPW
    c0 = pltpu.async_copy(idx_hbm.at[pl.ds(base, _BPW)], myidx, sem)
    c1 = pltpu.async_copy(updf_hbm.at[pl.ds(base, _BPW)], rows, sem)
    c2 = pltpu.async_copy(updm_hbm.at[pl.ds(base, _BPW)], vals, sem)
    c0.wait()
    c1.wait()
    c2.wait()
    s0 = pltpu.async_copy(rows, newf_ref.at[myidx], sem)
    s1 = pltpu.async_copy(vals, newm_ref.at[myidx], sem)
    s0.wait()
    s1.wait()


_sc_scatter = pl.kernel(
    _sc_scatter_body,
    out_type=(),
    mesh=plsc.VectorSubcoreMesh(core_axis_name="c", subcore_axis_name="s",
                                num_cores=2, num_subcores=16),
    compiler_params=pltpu.CompilerParams(needs_layout_passes=False),
    scratch_types=[
        pltpu.VMEM((_BPW,), jnp.int32),
        pltpu.VMEM((_BPW, N_FEAT), jnp.float32),
        pltpu.VMEM((_BPW,), jnp.float32),
        pltpu.SemaphoreType.DMA,
    ],
)


def kernel(inputs, indexes, IoU, features, mIoU):
    indexes = indexes.astype(jnp.int32)
    miou2 = mIoU.reshape(M_ROWS, M_COLS)
    updf, updm = _sc_update(inputs, indexes, IoU, features, mIoU)
    logits_t, newf_base, newm2 = _tc_call(inputs, features, miou2)
    logits = logits_t.T
    f_ref = jax.new_ref(newf_base)
    m_ref = jax.new_ref(newm2.reshape(N_MEM))
    _sc_scatter(indexes, updf, updm, f_ref, m_ref)
    return logits, f_ref[...], m_ref[...]
